# baseline (speedup 1.0000x reference)
import functools

import jax
import jax.numpy as jnp
from jax import lax
from jax.experimental import pallas as pl
from jax.experimental.pallas import tpu as pltpu
from jax.experimental.pallas import tpu_sc as plsc

N_USERS = 30000
N_ITEMS = 70000
N_NODES = N_USERS + N_ITEMS
DIM = 32
N_EDGES = 1600000
N_LAYERS = 3

NC = 2
NS = 16
LANES = 16
HDIM = DIM // NC

GROUP = 128
GPC = 8
CHUNK = GROUP * GPC
N_GROUPS = -(-N_EDGES // GROUP)
CHUNKS_PER_SUB = -(-N_GROUPS // (NS * GPC))
N_GROUPS_PAD = CHUNKS_PER_SUB * NS * GPC
E_PAD = N_GROUPS_PAD * GROUP
assert CHUNKS_PER_SUB % 2 == 0
PAIRS_PER_SUB = CHUNKS_PER_SUB // 2

ACC_ROWS = 100096
ACC_PER_TILE = ACC_ROWS // NS
ZERO_CHUNK = 512

DRAIN_CHUNK = 256
N_FULL_DRAIN = N_NODES // DRAIN_CHUNK
DRAIN_REM = N_NODES - N_FULL_DRAIN * DRAIN_CHUNK
DRAIN_ITERS = -(-(N_FULL_DRAIN + 1) // NS)

RING = 1024
NBUF = RING // GROUP
NFLY = 4
SUM_OFF = 512


def _phase(s, emb_hbm, sdw_hbm, cur_hbm, comb_hbm,
           sdw_v, rows_v, acc, sem, sem_s, sem_i):
    def init_copy(base, n):
        pltpu.sync_copy(emb_hbm.at[pl.ds(base, n)], rows_v.at[pl.ds(0, n)])
        pltpu.sync_copy(rows_v.at[pl.ds(0, n)], cur_hbm.at[pl.ds(base, n)])
        pltpu.sync_copy(rows_v.at[pl.ds(0, n)], comb_hbm.at[pl.ds(base, n)])

    n_full_init = N_NODES // ZERO_CHUNK
    init_rem = N_NODES - n_full_init * ZERO_CHUNK
    for q in range(-(-(n_full_init + 1) // NS)):
        ci = q * NS + s

        @pl.when(ci < n_full_init)
        def _():
            init_copy(ci * ZERO_CHUNK, ZERO_CHUNK)

        @pl.when(ci == n_full_init)
        def _():
            init_copy(n_full_init * ZERO_CHUNK, init_rem)

    def layer_body(lk, carry):
        scale = jnp.where(lk == N_LAYERS - 1, jnp.float32(1.0 / (N_LAYERS + 1)),
                          jnp.float32(1.0))
        _one_layer(lk, scale, s, sdw_hbm, cur_hbm, comb_hbm,
                   sdw_v, rows_v, acc, sem, sem_s, sem_i)
        return carry

    lax.fori_loop(0, N_LAYERS, layer_body, None)


def _one_layer(lk, scale, s, sdw_hbm, cur_hbm, comb_hbm,
               sdw_v, rows_v, acc, sem, sem_s, sem_i):
    del lk
    emb_hbm = cur_hbm

    def zero_body(e, _):
        rows_v[e, pl.ds(0, LANES)] = jnp.zeros((LANES,), jnp.float32)
        return _
    lax.fori_loop(0, ZERO_CHUNK, zero_body, None)

    tb = s * ACC_PER_TILE
    off = 0
    while off < ACC_PER_TILE:
        n = min(ZERO_CHUNK, ACC_PER_TILE - off)
        pltpu.sync_copy(rows_v.at[pl.ds(0, n)],
                        acc.at[pl.ds(tb + off, n)])
        off += n

    plsc.subcore_barrier()

    def chunk_rows(t):
        return (t * NS + s) * GPC

    def process_chunk(t, h, first, maybe_last_prefetch):
        if not first:
            pltpu.make_async_copy(sdw_hbm.at[pl.ds(0, GPC)],
                                  sdw_v.at[h], sem_i).wait()

        def prefetch():
            pltpu.async_copy(sdw_hbm.at[pl.ds(chunk_rows(t + 1), GPC)],
                             sdw_v.at[1 - h], sem_i)
        if maybe_last_prefetch is None:
            prefetch()
        else:
            pl.when(maybe_last_prefetch)(prefetch)

        def fire_gather(j):
            return pltpu.async_copy(
                emb_hbm.at[sdw_v.at[h, j, 0]],
                rows_v.at[pl.ds((j % NBUF) * GROUP, GROUP)], sem)

        def fire_scatter(j):
            return pltpu.async_copy(
                rows_v.at[pl.ds((j % NBUF) * GROUP, GROUP)],
                acc.at[sdw_v.at[h, j, 1]], sem_s, add=True)

        gathers = [None] * GPC
        scatters = [None] * GPC
        for j in range(NFLY):
            gathers[j] = fire_gather(j)
        for j in range(GPC):
            nj = j + NFLY
            if nj < GPC:
                if nj >= NBUF:
                    scatters[nj - NBUF].wait()
                gathers[nj] = fire_gather(nj)
            gathers[j].wait()

            slot = (j % NBUF) * GROUP

            def mul_body(i, _):
                wvec = jax.lax.bitcast_convert_type(
                    sdw_v[h, j, 2, pl.ds(i * LANES, LANES)], jnp.float32)
                for k in range(LANES):
                    wv = wvec[k]
                    e = slot + i * LANES + k
                    rows_v[e, pl.ds(0, LANES)] = (
                        rows_v[e, pl.ds(0, LANES)] * wv)
                return _
            lax.fori_loop(0, GROUP // LANES, mul_body, None)

            scatters[j] = fire_scatter(j)
        for j in range(GPC - NBUF, GPC):
            scatters[j].wait()

    pltpu.sync_copy(sdw_hbm.at[pl.ds(chunk_rows(0), GPC)], sdw_v.at[0])

    def pair_body(t2, _):
        t0 = t2 * 2
        process_chunk(t0, 0, False, None)
        process_chunk(t0 + 1, 1, False, t2 < PAIRS_PER_SUB - 1)
        return _

    process_chunk(0, 0, True, None)
    process_chunk(1, 1, False, None)
    lax.fori_loop(1, PAIRS_PER_SUB, pair_body, None)

    plsc.subcore_barrier()

    def drain(base, n):
        pltpu.sync_copy(acc.at[pl.ds(base, n)], rows_v.at[pl.ds(0, n)])
        pltpu.sync_copy(comb_hbm.at[pl.ds(base, n)],
                        rows_v.at[pl.ds(SUM_OFF, n)])

        def out_body(e, _):
            a0 = rows_v[e, pl.ds(0, LANES)]
            es = SUM_OFF + e
            s0 = (rows_v[es, pl.ds(0, LANES)] + a0) * scale
            rows_v[es, pl.ds(0, LANES)] = s0
            return _
        lax.fori_loop(0, n, out_body, None)

        pltpu.sync_copy(rows_v.at[pl.ds(SUM_OFF, n)],
                        comb_hbm.at[pl.ds(base, n)])
        pltpu.sync_copy(rows_v.at[pl.ds(0, n)], cur_hbm.at[pl.ds(base, n)])

    for q in range(DRAIN_ITERS):
        ci = q * NS + s

        @pl.when(ci < N_FULL_DRAIN)
        def _():
            drain(ci * DRAIN_CHUNK, DRAIN_CHUNK)

        @pl.when(ci == N_FULL_DRAIN)
        def _():
            drain(N_FULL_DRAIN * DRAIN_CHUNK, DRAIN_REM)

    plsc.subcore_barrier()


def _full_body(emb_lo, emb_hi, sdw_hbm,
               cur_lo, cur_hi, comb_lo, comb_hi,
               sdw_v, rows_v, acc, sem, sem_s, sem_i):
    c = lax.axis_index("c")
    s = lax.axis_index("s")

    @pl.when(c == 0)
    def _():
        _phase(s, emb_lo, sdw_hbm, cur_lo, comb_lo,
               sdw_v, rows_v, acc, sem, sem_s, sem_i)

    @pl.when(c == 1)
    def _():
        _phase(s, emb_hi, sdw_hbm, cur_hi, comb_hi,
               sdw_v, rows_v, acc, sem, sem_s, sem_i)


@functools.lru_cache(maxsize=None)
def _full_kernel():
    mesh = plsc.VectorSubcoreMesh(core_axis_name="c", subcore_axis_name="s")

    half = jax.ShapeDtypeStruct((N_NODES, HDIM), jnp.float32)
    return pl.kernel(
        _full_body,
        out_type=(half, half, half, half),
        mesh=mesh,
        compiler_params=pltpu.CompilerParams(use_tc_tiling_on_sc=False),
        scratch_types=[
            pltpu.VMEM((2, GPC, 3, GROUP), jnp.int32),
            pltpu.VMEM((RING, HDIM), jnp.float32),
            pltpu.VMEM_SHARED((ACC_ROWS, HDIM), jnp.float32),
            pltpu.SemaphoreType.DMA,
            pltpu.SemaphoreType.DMA,
            pltpu.SemaphoreType.DMA,
        ],
    )


def kernel(user_emb, item_emb, edge_weight, edge_index):
    all_emb = jnp.concatenate([user_emb, item_emb], axis=0)
    emb_lo = all_emb[:, :HDIM]
    emb_hi = all_emb[:, HDIM:]
    src = edge_index[0].astype(jnp.int32)
    dst = edge_index[1].astype(jnp.int32)
    w = edge_weight.astype(jnp.float32)

    pad = E_PAD - N_EDGES
    src = jnp.concatenate([src, jnp.zeros((pad,), jnp.int32)])
    dst = jnp.concatenate([dst, jnp.zeros((pad,), jnp.int32)])
    w = jnp.concatenate([w, jnp.zeros((pad,), jnp.float32)])
    wbits = jax.lax.bitcast_convert_type(w, jnp.int32)
    sdw = jnp.stack([src.reshape(N_GROUPS_PAD, GROUP),
                     dst.reshape(N_GROUPS_PAD, GROUP),
                     wbits.reshape(N_GROUPS_PAD, GROUP)], axis=1)

    _, _, comb_lo, comb_hi = _full_kernel()(emb_lo, emb_hi, sdw)

    comb = jnp.concatenate([comb_lo, comb_hi], axis=1)
    return comb[:N_USERS], comb[N_USERS:]


# speedup vs baseline: 1.0498x; 1.0498x over previous
import functools

import jax
import jax.numpy as jnp
from jax import lax
from jax.experimental import pallas as pl
from jax.experimental.pallas import tpu as pltpu
from jax.experimental.pallas import tpu_sc as plsc

N_USERS = 30000
N_ITEMS = 70000
N_NODES = N_USERS + N_ITEMS
DIM = 32
N_EDGES = 1600000
N_LAYERS = 3

NC = 2
NS = 16
LANES = 16
HDIM = DIM // NC

GROUP = 128
GPC = 8
CHUNK = GROUP * GPC
N_GROUPS = -(-N_EDGES // GROUP)
CHUNKS_PER_SUB = -(-N_GROUPS // (NS * GPC))
N_GROUPS_PAD = CHUNKS_PER_SUB * NS * GPC
E_PAD = N_GROUPS_PAD * GROUP
assert CHUNKS_PER_SUB % 2 == 0
PAIRS_PER_SUB = CHUNKS_PER_SUB // 2

ACC_ROWS = 100096
ACC_PER_TILE = ACC_ROWS // NS
ZERO_CHUNK = 512

DRAIN_CHUNK = 256
N_FULL_DRAIN = N_NODES // DRAIN_CHUNK
DRAIN_REM = N_NODES - N_FULL_DRAIN * DRAIN_CHUNK
DRAIN_ITERS = -(-(N_FULL_DRAIN + 1) // NS)

RING = 1024
NBUF = RING // GROUP
NFLY = 6
SUM_OFF = 512


def _phase(s, emb_hbm, sdw_hbm, cur_hbm, comb_hbm,
           sdw_v, rows_v, acc, sem, sem_s, sem_i):
    def init_copy(base, n):
        pltpu.sync_copy(emb_hbm.at[pl.ds(base, n)], rows_v.at[pl.ds(0, n)])
        pltpu.sync_copy(rows_v.at[pl.ds(0, n)], cur_hbm.at[pl.ds(base, n)])
        pltpu.sync_copy(rows_v.at[pl.ds(0, n)], comb_hbm.at[pl.ds(base, n)])

    n_full_init = N_NODES // ZERO_CHUNK
    init_rem = N_NODES - n_full_init * ZERO_CHUNK
    for q in range(-(-(n_full_init + 1) // NS)):
        ci = q * NS + s

        @pl.when(ci < n_full_init)
        def _():
            init_copy(ci * ZERO_CHUNK, ZERO_CHUNK)

        @pl.when(ci == n_full_init)
        def _():
            init_copy(n_full_init * ZERO_CHUNK, init_rem)

    def layer_body(lk, carry):
        scale = jnp.where(lk == N_LAYERS - 1, jnp.float32(1.0 / (N_LAYERS + 1)),
                          jnp.float32(1.0))
        _one_layer(lk, scale, s, sdw_hbm, cur_hbm, comb_hbm,
                   sdw_v, rows_v, acc, sem, sem_s, sem_i)
        return carry

    lax.fori_loop(0, N_LAYERS, layer_body, None)


def _one_layer(lk, scale, s, sdw_hbm, cur_hbm, comb_hbm,
               sdw_v, rows_v, acc, sem, sem_s, sem_i):
    del lk
    emb_hbm = cur_hbm

    def zero_body(e, _):
        rows_v[e, pl.ds(0, LANES)] = jnp.zeros((LANES,), jnp.float32)
        return _
    lax.fori_loop(0, ZERO_CHUNK, zero_body, None)

    tb = s * ACC_PER_TILE
    off = 0
    while off < ACC_PER_TILE:
        n = min(ZERO_CHUNK, ACC_PER_TILE - off)
        pltpu.sync_copy(rows_v.at[pl.ds(0, n)],
                        acc.at[pl.ds(tb + off, n)])
        off += n

    plsc.subcore_barrier()

    def chunk_rows(t):
        return (t * NS + s) * GPC

    def process_chunk(t, h, first, maybe_last_prefetch):
        if not first:
            pltpu.make_async_copy(sdw_hbm.at[pl.ds(0, GPC)],
                                  sdw_v.at[h], sem_i).wait()

        def prefetch():
            pltpu.async_copy(sdw_hbm.at[pl.ds(chunk_rows(t + 1), GPC)],
                             sdw_v.at[1 - h], sem_i)
        if maybe_last_prefetch is None:
            prefetch()
        else:
            pl.when(maybe_last_prefetch)(prefetch)

        def fire_gather(j):
            return pltpu.async_copy(
                emb_hbm.at[sdw_v.at[h, j, 0]],
                rows_v.at[pl.ds((j % NBUF) * GROUP, GROUP)], sem)

        def fire_scatter(j):
            return pltpu.async_copy(
                rows_v.at[pl.ds((j % NBUF) * GROUP, GROUP)],
                acc.at[sdw_v.at[h, j, 1]], sem_s, add=True)

        gathers = [None] * GPC
        scatters = [None] * GPC
        for j in range(NFLY):
            gathers[j] = fire_gather(j)
        for j in range(GPC):
            nj = j + NFLY
            if nj < GPC:
                if nj >= NBUF:
                    scatters[nj - NBUF].wait()
                gathers[nj] = fire_gather(nj)
            gathers[j].wait()

            slot = (j % NBUF) * GROUP

            def mul_body(i, _):
                wvec = jax.lax.bitcast_convert_type(
                    sdw_v[h, j, 2, pl.ds(i * LANES, LANES)], jnp.float32)
                for k in range(LANES):
                    wv = wvec[k]
                    e = slot + i * LANES + k
                    rows_v[e, pl.ds(0, LANES)] = (
                        rows_v[e, pl.ds(0, LANES)] * wv)
                return _
            lax.fori_loop(0, GROUP // LANES, mul_body, None)

            scatters[j] = fire_scatter(j)
        for j in range(GPC - NBUF, GPC):
            scatters[j].wait()

    pltpu.sync_copy(sdw_hbm.at[pl.ds(chunk_rows(0), GPC)], sdw_v.at[0])

    def pair_body(t2, _):
        t0 = t2 * 2
        process_chunk(t0, 0, False, None)
        process_chunk(t0 + 1, 1, False, t2 < PAIRS_PER_SUB - 1)
        return _

    process_chunk(0, 0, True, None)
    process_chunk(1, 1, False, None)
    lax.fori_loop(1, PAIRS_PER_SUB, pair_body, None)

    plsc.subcore_barrier()

    def drain(base, n):
        pltpu.sync_copy(acc.at[pl.ds(base, n)], rows_v.at[pl.ds(0, n)])
        pltpu.sync_copy(comb_hbm.at[pl.ds(base, n)],
                        rows_v.at[pl.ds(SUM_OFF, n)])

        def out_body(e, _):
            a0 = rows_v[e, pl.ds(0, LANES)]
            es = SUM_OFF + e
            s0 = (rows_v[es, pl.ds(0, LANES)] + a0) * scale
            rows_v[es, pl.ds(0, LANES)] = s0
            return _
        lax.fori_loop(0, n, out_body, None)

        pltpu.sync_copy(rows_v.at[pl.ds(SUM_OFF, n)],
                        comb_hbm.at[pl.ds(base, n)])
        pltpu.sync_copy(rows_v.at[pl.ds(0, n)], cur_hbm.at[pl.ds(base, n)])

    for q in range(DRAIN_ITERS):
        ci = q * NS + s

        @pl.when(ci < N_FULL_DRAIN)
        def _():
            drain(ci * DRAIN_CHUNK, DRAIN_CHUNK)

        @pl.when(ci == N_FULL_DRAIN)
        def _():
            drain(N_FULL_DRAIN * DRAIN_CHUNK, DRAIN_REM)

    plsc.subcore_barrier()


def _full_body(emb_lo, emb_hi, sdw_hbm,
               cur_lo, cur_hi, comb_lo, comb_hi,
               sdw_v, rows_v, acc, sem, sem_s, sem_i):
    c = lax.axis_index("c")
    s = lax.axis_index("s")

    @pl.when(c == 0)
    def _():
        _phase(s, emb_lo, sdw_hbm, cur_lo, comb_lo,
               sdw_v, rows_v, acc, sem, sem_s, sem_i)

    @pl.when(c == 1)
    def _():
        _phase(s, emb_hi, sdw_hbm, cur_hi, comb_hi,
               sdw_v, rows_v, acc, sem, sem_s, sem_i)


@functools.lru_cache(maxsize=None)
def _full_kernel():
    mesh = plsc.VectorSubcoreMesh(core_axis_name="c", subcore_axis_name="s")

    half = jax.ShapeDtypeStruct((N_NODES, HDIM), jnp.float32)
    return pl.kernel(
        _full_body,
        out_type=(half, half, half, half),
        mesh=mesh,
        compiler_params=pltpu.CompilerParams(use_tc_tiling_on_sc=False),
        scratch_types=[
            pltpu.VMEM((2, GPC, 3, GROUP), jnp.int32),
            pltpu.VMEM((RING, HDIM), jnp.float32),
            pltpu.VMEM_SHARED((ACC_ROWS, HDIM), jnp.float32),
            pltpu.SemaphoreType.DMA,
            pltpu.SemaphoreType.DMA,
            pltpu.SemaphoreType.DMA,
        ],
    )


def kernel(user_emb, item_emb, edge_weight, edge_index):
    all_emb = jnp.concatenate([user_emb, item_emb], axis=0)
    emb_lo = all_emb[:, :HDIM]
    emb_hi = all_emb[:, HDIM:]
    src = edge_index[0].astype(jnp.int32)
    dst = edge_index[1].astype(jnp.int32)
    w = edge_weight.astype(jnp.float32)

    pad = E_PAD - N_EDGES
    src = jnp.concatenate([src, jnp.zeros((pad,), jnp.int32)])
    dst = jnp.concatenate([dst, jnp.zeros((pad,), jnp.int32)])
    w = jnp.concatenate([w, jnp.zeros((pad,), jnp.float32)])
    wbits = jax.lax.bitcast_convert_type(w, jnp.int32)
    sdw = jnp.stack([src.reshape(N_GROUPS_PAD, GROUP),
                     dst.reshape(N_GROUPS_PAD, GROUP),
                     wbits.reshape(N_GROUPS_PAD, GROUP)], axis=1)

    _, _, comb_lo, comb_hi = _full_kernel()(emb_lo, emb_hi, sdw)

    comb = jnp.concatenate([comb_lo, comb_hi], axis=1)
    return comb[:N_USERS], comb[N_USERS:]
